# baseline (speedup 1.0000x reference)
import functools

import jax
import jax.numpy as jnp
from jax import lax
from jax.experimental import pallas as pl
from jax.experimental.pallas import tpu as pltpu
from jax.experimental.pallas import tpu_sc as plsc

B = 4096
T = 200
NPLANE = 3
ROWS = T
COLS = B
TROW = 8
QW = COLS // 8
L = 16
NU = (ROWS // TROW) * 8


def kernel(observations, norm_factors):
    xt = jnp.transpose(observations, (2, 1, 0))

    mesh = plsc.VectorSubcoreMesh(core_axis_name="c", subcore_axis_name="s")

    @functools.partial(
        pl.kernel,
        mesh=mesh,
        out_type=jax.ShapeDtypeStruct((NPLANE, ROWS, COLS), jnp.float32),
        compiler_params=pltpu.CompilerParams(needs_layout_passes=False),
        scratch_types=[
            pltpu.VMEM((256,), jnp.float32),
            pltpu.VMEM((256,), jnp.float32),
            pltpu.VMEM((TROW, QW), jnp.int32),
            pltpu.VMEM((TROW, QW), jnp.int32),
            pltpu.VMEM((TROW, QW), jnp.int32),
            pltpu.VMEM((TROW, QW), jnp.int32),
            pltpu.VMEM((TROW, QW), jnp.float32),
            pltpu.VMEM((TROW, QW), jnp.float32),
            pltpu.VMEM((TROW, QW), jnp.float32),
            pltpu.VMEM((TROW, QW), jnp.float32),
            pltpu.SemaphoreType.DMA,
            pltpu.SemaphoreType.DMA,
            pltpu.SemaphoreType.DMA,
            pltpu.SemaphoreType.DMA,
        ],
    )
    def sc_kernel(x_hbm, nf_hbm, out_hbm, nf_v, rcp_v,
                  i1a, i1b, i2a, i2b, o1a, o1b, o2a, o2b,
                  in_sa, in_sb, out_sa, out_sb):
        w = lax.axis_index("s") * 2 + lax.axis_index("c")

        nj = jnp.where(w < 8, 7, 6)
        base_j = jnp.where(w < 8, 7 * w, 56 + 6 * (w - 8))
        nc = jnp.where(w < 8, 4, 7)
        base_c = jnp.where(w < 8, 4 * w, 32 + 7 * (w - 8))

        i1 = (i1a, i1b)
        i2 = (i2a, i2b)
        o1 = (o1a, o1b)
        o2 = (o2a, o2b)
        in_s = (in_sa, in_sb)
        out_s = (out_sa, out_sb)

        def slices(u):
            return pl.ds((u // 8) * TROW, TROW), pl.ds((u % 8) * QW, QW)

        def in_joint(u, s):
            rs, cs = slices(u)
            return (pltpu.make_async_copy(x_hbm.at[1, rs, cs], i1[s], in_s[s]),
                    pltpu.make_async_copy(x_hbm.at[2, rs, cs], i2[s], in_s[s]))

        def out_joint(u, s):
            rs, cs = slices(u)
            return (pltpu.make_async_copy(o1[s], out_hbm.at[1, rs, cs], out_s[s]),
                    pltpu.make_async_copy(o2[s], out_hbm.at[2, rs, cs], out_s[s]))

        def in_cast(u, s):
            rs, cs = slices(u)
            return (pltpu.make_async_copy(x_hbm.at[0, rs, cs], i1[s], in_s[s]),)

        def out_cast(u, s):
            rs, cs = slices(u)
            return (pltpu.make_async_copy(o1[s], out_hbm.at[0, rs, cs], out_s[s]),)

        def start(copies):
            for c in copies:
                c.start()

        def wait(copies):
            for c in copies:
                c.wait()

        def compute_joint(s):
            i1s, i2s, o1s, o2s = i1[s], i2[s], o1[s], o2[s]

            def row_body(r, carry):
                @plsc.parallel_loop(0, QW // L, unroll=8)
                def _(j):
                    col = pl.ds(j * L, L)
                    a = i1s[r, col]
                    v = i2s[r, col]
                    rcp = plsc.load_gather(rcp_v, [a])
                    o1s[r, col] = a.astype(jnp.float32)
                    o2s[r, col] = v.astype(jnp.float32) * rcp
                return carry

            lax.fori_loop(0, TROW, row_body, 0)

        def compute_cast(s):
            i1s, o1s = i1[s], o1[s]

            def row_body(r, carry):
                @plsc.parallel_loop(0, QW // L, unroll=8)
                def _(j):
                    col = pl.ds(j * L, L)
                    o1s[r, col] = i1s[r, col].astype(jnp.float32)
                return carry

            lax.fori_loop(0, TROW, row_body, 0)

        start(in_joint(base_j + 0, 0))
        start(in_joint(base_j + 1, 1))

        pltpu.sync_copy(nf_hbm, nf_v)
        for i in range(256 // L):
            rcp_v[pl.ds(i * L, L)] = 1.0 / nf_v[pl.ds(i * L, L)]

        for q in range(7):
            s = q & 1
            u = base_j + q

            def iter_body(u=u, s=s, q=q):
                wait(in_joint(u, s))
                if q >= 2:
                    wait(out_joint(base_j + (q - 2), s))
                compute_joint(s)
                start(out_joint(u, s))
                if q + 2 < 7:
                    @pl.when(q + 2 < nj)
                    def _():
                        start(in_joint(base_j + (q + 2), s))

            if q < 6:
                iter_body()
            else:
                pl.when(q < nj)(iter_body)

        start(in_cast(base_c + 0, 0))
        start(in_cast(base_c + 1, 1))

        wait(out_joint(base_j, 0))
        wait(out_joint(base_j, 1))

        for q in range(7):
            s = q & 1
            u = base_c + q

            def iter_body(u=u, s=s, q=q):
                wait(in_cast(u, s))
                if q >= 2:
                    wait(out_cast(base_c + (q - 2), s))
                compute_cast(s)
                start(out_cast(u, s))
                if q + 2 < 7:
                    @pl.when(q + 2 < nc)
                    def _():
                        start(in_cast(base_c + (q + 2), s))

            if q < 4:
                iter_body()
            else:
                pl.when(q < nc)(iter_body)

        wait(out_cast(base_c, 0))
        wait(out_cast(base_c, 1))

    ot = sc_kernel(xt, norm_factors)
    return jnp.transpose(ot, (2, 1, 0))


# speedup vs baseline: 1.0619x; 1.0619x over previous
import functools

import jax
import jax.numpy as jnp
from jax import lax
from jax.experimental import pallas as pl
from jax.experimental.pallas import tpu as pltpu
from jax.experimental.pallas import tpu_sc as plsc

B = 4096
T = 200
NPLANE = 3
ROWS = T
COLS = B
TROW = 8
QW = COLS // 4
L = 16


def kernel(observations, norm_factors):
    xt = jnp.transpose(observations, (2, 1, 0))

    mesh = plsc.VectorSubcoreMesh(core_axis_name="c", subcore_axis_name="s")

    @functools.partial(
        pl.kernel,
        mesh=mesh,
        out_type=jax.ShapeDtypeStruct((NPLANE, ROWS, COLS), jnp.float32),
        compiler_params=pltpu.CompilerParams(needs_layout_passes=False),
        scratch_types=[
            pltpu.VMEM((256,), jnp.float32),
            pltpu.VMEM((256,), jnp.float32),
            pltpu.VMEM((TROW, QW), jnp.int32),
            pltpu.VMEM((TROW, QW), jnp.int32),
            pltpu.VMEM((TROW, QW), jnp.int32),
            pltpu.VMEM((TROW, QW), jnp.int32),
            pltpu.VMEM((TROW, QW), jnp.float32),
            pltpu.VMEM((TROW, QW), jnp.float32),
            pltpu.VMEM((TROW, QW), jnp.float32),
            pltpu.VMEM((TROW, QW), jnp.float32),
            pltpu.SemaphoreType.DMA,
            pltpu.SemaphoreType.DMA,
            pltpu.SemaphoreType.DMA,
            pltpu.SemaphoreType.DMA,
        ],
    )
    def sc_kernel(x_hbm, nf_hbm, out_hbm, nf_v, rcp_v,
                  i1a, i1b, i2a, i2b, o1a, o1b, o2a, o2b,
                  in_sa, in_sb, out_sa, out_sb):
        w = lax.axis_index("s") * 2 + lax.axis_index("c")

        nj = jnp.where(w < 4, 4, 3)
        base_j = jnp.where(w < 4, 4 * w, 16 + 3 * (w - 4))
        nc = jnp.where(w < 4, 1, jnp.where(w < 16, 4, 3))
        base_c = jnp.where(w < 4, w,
                           jnp.where(w < 16, 4 + 4 * (w - 4), 52 + 3 * (w - 16)))

        i1 = (i1a, i1b)
        i2 = (i2a, i2b)
        o1 = (o1a, o1b)
        o2 = (o2a, o2b)
        in_s = (in_sa, in_sb)
        out_s = (out_sa, out_sb)

        def slices(u):
            return pl.ds((u // 4) * TROW, TROW), pl.ds((u % 4) * QW, QW)

        def in_joint(u, s):
            rs, cs = slices(u)
            return (pltpu.make_async_copy(x_hbm.at[1, rs, cs], i1[s], in_s[s]),
                    pltpu.make_async_copy(x_hbm.at[2, rs, cs], i2[s], in_s[s]))

        def out_joint(u, s):
            rs, cs = slices(u)
            return (pltpu.make_async_copy(o1[s], out_hbm.at[1, rs, cs], out_s[s]),
                    pltpu.make_async_copy(o2[s], out_hbm.at[2, rs, cs], out_s[s]))

        def in_cast(u, s):
            rs, cs = slices(u)
            return (pltpu.make_async_copy(x_hbm.at[0, rs, cs], i1[s], in_s[s]),)

        def out_cast(u, s):
            rs, cs = slices(u)
            return (pltpu.make_async_copy(o1[s], out_hbm.at[0, rs, cs], out_s[s]),)

        def start(copies):
            for c in copies:
                c.start()

        def wait(copies):
            for c in copies:
                c.wait()

        def compute_joint(s):
            i1s, i2s, o1s, o2s = i1[s], i2[s], o1[s], o2[s]

            def row_body(r, carry):
                @plsc.parallel_loop(0, QW // L, unroll=8)
                def _(j):
                    col = pl.ds(j * L, L)
                    a = i1s[r, col]
                    v = i2s[r, col]
                    rcp = plsc.load_gather(rcp_v, [a])
                    o1s[r, col] = a.astype(jnp.float32)
                    o2s[r, col] = v.astype(jnp.float32) * rcp
                return carry

            lax.fori_loop(0, TROW, row_body, 0)

        def compute_cast(s):
            i1s, o1s = i1[s], o1[s]

            def row_body(r, carry):
                @plsc.parallel_loop(0, QW // L, unroll=8)
                def _(j):
                    col = pl.ds(j * L, L)
                    o1s[r, col] = i1s[r, col].astype(jnp.float32)
                return carry

            lax.fori_loop(0, TROW, row_body, 0)

        start(in_joint(base_j + 0, 0))
        start(in_joint(base_j + 1, 1))

        pltpu.sync_copy(nf_hbm, nf_v)
        for i in range(256 // L):
            rcp_v[pl.ds(i * L, L)] = 1.0 / nf_v[pl.ds(i * L, L)]

        for q in range(4):
            s = q & 1
            u = base_j + q

            def iter_body(u=u, s=s, q=q):
                wait(in_joint(u, s))
                if q >= 2:
                    wait(out_joint(base_j + (q - 2), s))
                compute_joint(s)
                start(out_joint(u, s))
                if q + 2 < 4:
                    @pl.when(q + 2 < nj)
                    def _():
                        start(in_joint(base_j + (q + 2), s))

            if q < 3:
                iter_body()
            else:
                pl.when(q < nj)(iter_body)

        start(in_cast(base_c + 0, 0))
        @pl.when(nc >= 2)
        def _():
            start(in_cast(base_c + 1, 1))

        wait(out_joint(base_j, 0))
        wait(out_joint(base_j, 1))

        for q in range(4):
            s = q & 1
            u = base_c + q

            def iter_body(u=u, s=s, q=q):
                wait(in_cast(u, s))
                if q >= 2:
                    wait(out_cast(base_c + (q - 2), s))
                compute_cast(s)
                start(out_cast(u, s))
                if q + 2 < 4:
                    @pl.when(q + 2 < nc)
                    def _():
                        start(in_cast(base_c + (q + 2), s))

            if q < 1:
                iter_body()
            else:
                pl.when(q < nc)(iter_body)

        wait(out_cast(base_c, 0))
        @pl.when(nc >= 2)
        def _():
            wait(out_cast(base_c, 1))

    ot = sc_kernel(xt, norm_factors)
    return jnp.transpose(ot, (2, 1, 0))


# speedup vs baseline: 1.0671x; 1.0049x over previous
import functools

import jax
import jax.numpy as jnp
from jax import lax
from jax.experimental import pallas as pl
from jax.experimental.pallas import tpu as pltpu
from jax.experimental.pallas import tpu_sc as plsc

B = 4096
T = 200
NPLANE = 3
ROWS = T
COLS = B
TROW = 8
QW = COLS // 4
L = 16


def kernel(observations, norm_factors):
    xt = jnp.transpose(observations, (2, 1, 0))

    mesh = plsc.VectorSubcoreMesh(core_axis_name="c", subcore_axis_name="s")

    @functools.partial(
        pl.kernel,
        mesh=mesh,
        out_type=jax.ShapeDtypeStruct((NPLANE, ROWS, COLS), jnp.float32),
        compiler_params=pltpu.CompilerParams(needs_layout_passes=False),
        scratch_types=[
            pltpu.VMEM((256,), jnp.float32),
            pltpu.VMEM((256,), jnp.float32),
            pltpu.VMEM((2, TROW, QW), jnp.int32),
            pltpu.VMEM((2, TROW, QW), jnp.int32),
            pltpu.VMEM((2, TROW, QW), jnp.float32),
            pltpu.VMEM((2, TROW, QW), jnp.float32),
            pltpu.SemaphoreType.DMA,
            pltpu.SemaphoreType.DMA,
            pltpu.SemaphoreType.DMA,
            pltpu.SemaphoreType.DMA,
        ],
    )
    def sc_kernel(x_hbm, nf_hbm, out_hbm, nf_v, rcp_v,
                  ia, ib, oa, ob,
                  in_sa, in_sb, out_sa, out_sb):
        w = lax.axis_index("s") * 2 + lax.axis_index("c")

        nj = jnp.where(w < 4, 4, 3)
        base_j = jnp.where(w < 4, 4 * w, 16 + 3 * (w - 4))
        nc = jnp.where(w < 4, 1, jnp.where(w < 16, 4, 3))
        base_c = jnp.where(w < 4, w,
                           jnp.where(w < 16, 4 + 4 * (w - 4), 52 + 3 * (w - 16)))

        ibuf = (ia, ib)
        obuf = (oa, ob)
        in_s = (in_sa, in_sb)
        out_s = (out_sa, out_sb)

        def slices(u):
            return pl.ds((u // 4) * TROW, TROW), pl.ds((u % 4) * QW, QW)

        def in_joint(u, s):
            rs, cs = slices(u)
            return (pltpu.make_async_copy(
                x_hbm.at[pl.ds(1, 2), rs, cs], ibuf[s], in_s[s]),)

        def out_joint(u, s):
            rs, cs = slices(u)
            return (pltpu.make_async_copy(
                obuf[s], out_hbm.at[pl.ds(1, 2), rs, cs], out_s[s]),)

        def in_cast(u, s):
            rs, cs = slices(u)
            return (pltpu.make_async_copy(
                x_hbm.at[0, rs, cs], ibuf[s].at[0], in_s[s]),)

        def out_cast(u, s):
            rs, cs = slices(u)
            return (pltpu.make_async_copy(
                obuf[s].at[0], out_hbm.at[0, rs, cs], out_s[s]),)

        def start(copies):
            for c in copies:
                c.start()

        def wait(copies):
            for c in copies:
                c.wait()

        def compute_joint(s):
            is_, os_ = ibuf[s], obuf[s]

            def row_body(r, carry):
                @plsc.parallel_loop(0, QW // L, unroll=8)
                def _(j):
                    col = pl.ds(j * L, L)
                    a = is_[0, r, col]
                    v = is_[1, r, col]
                    rcp = plsc.load_gather(rcp_v, [a])
                    os_[0, r, col] = a.astype(jnp.float32)
                    os_[1, r, col] = v.astype(jnp.float32) * rcp
                return carry

            lax.fori_loop(0, TROW, row_body, 0)

        def compute_cast(s):
            is_, os_ = ibuf[s], obuf[s]

            def row_body(r, carry):
                @plsc.parallel_loop(0, QW // L, unroll=8)
                def _(j):
                    col = pl.ds(j * L, L)
                    os_[0, r, col] = is_[0, r, col].astype(jnp.float32)
                return carry

            lax.fori_loop(0, TROW, row_body, 0)

        start(in_joint(base_j + 0, 0))
        start(in_joint(base_j + 1, 1))

        pltpu.sync_copy(nf_hbm, nf_v)
        for i in range(256 // L):
            rcp_v[pl.ds(i * L, L)] = 1.0 / nf_v[pl.ds(i * L, L)]

        for q in range(4):
            s = q & 1
            u = base_j + q

            def iter_body(u=u, s=s, q=q):
                wait(in_joint(u, s))
                if q >= 2:
                    wait(out_joint(base_j + (q - 2), s))
                compute_joint(s)
                start(out_joint(u, s))
                if q + 2 < 4:
                    @pl.when(q + 2 < nj)
                    def _():
                        start(in_joint(base_j + (q + 2), s))

            if q < 3:
                iter_body()
            else:
                pl.when(q < nj)(iter_body)

        start(in_cast(base_c + 0, 0))
        @pl.when(nc >= 2)
        def _():
            start(in_cast(base_c + 1, 1))

        wait(out_joint(base_j, 0))
        wait(out_joint(base_j, 1))

        for q in range(4):
            s = q & 1
            u = base_c + q

            def iter_body(u=u, s=s, q=q):
                wait(in_cast(u, s))
                if q >= 2:
                    wait(out_cast(base_c + (q - 2), s))
                compute_cast(s)
                start(out_cast(u, s))
                if q + 2 < 4:
                    @pl.when(q + 2 < nc)
                    def _():
                        start(in_cast(base_c + (q + 2), s))

            if q < 1:
                iter_body()
            else:
                pl.when(q < nc)(iter_body)

        wait(out_cast(base_c, 0))
        @pl.when(nc >= 2)
        def _():
            wait(out_cast(base_c, 1))

    ot = sc_kernel(xt, norm_factors)
    return jnp.transpose(ot, (2, 1, 0))
